# Initial kernel scaffold; baseline (speedup 1.0000x reference)
import jax
import jax.numpy as jnp
from jax.experimental import pallas as pl


def kernel(x, edge_index, batch, params, Wc, bc):
    raise NotImplementedError("write your pallas kernel here")


# speedup vs baseline: 5.6820x; 5.6820x over previous
import functools

import jax
import jax.numpy as jnp
from jax import lax
from jax.experimental import pallas as pl
from jax.experimental.pallas import tpu as pltpu
from jax.experimental.pallas import tpu_sc as plsc

N = 10000
E = 320000
NG = 64
ACC_ROWS = 10240
NC, NS = 2, 16
NW = NC * NS
K = 128
ROWS_PER_W = ACC_ROWS // NW
E_SELF = E + N
E_PAD = ((E_SELF + NW * K - 1) // (NW * K)) * (NW * K)
E_ALL = E_PAD + K

_BLOCK_CONFIG = (3, 3, 3, 3)


def _extract32(vref, pos):
    li = lax.broadcasted_iota(jnp.int32, (16,), 0)
    lo = vref[pl.ds(0, 16)]
    hi = vref[pl.ds(16, 16)]
    p0 = jnp.where(li == pos, lo, 0)
    p1 = jnp.where(li + 16 == pos, hi, 0)
    return jnp.sum(p0) + jnp.sum(p1)


@functools.lru_cache(maxsize=None)
def _seg_kernel(W):
    mesh = plsc.VectorSubcoreMesh(core_axis_name="c", subcore_axis_name="s")
    acc_per_tile = ACC_ROWS // NS

    @functools.partial(
        pl.kernel,
        mesh=mesh,
        out_type=jax.ShapeDtypeStruct((NC, ACC_ROWS, W), jnp.float32),
        compiler_params=pltpu.CompilerParams(use_tc_tiling_on_sc=False,
                                             needs_layout_passes=False),
        scratch_types=[
            pltpu.VMEM((K,), jnp.int32),
            pltpu.VMEM((K,), jnp.int32),
            pltpu.VMEM((32,), jnp.int32),
            pltpu.VMEM((32,), jnp.int32),
            pltpu.VMEM((32,), jnp.int32),
            pltpu.VMEM((K, W), jnp.float32),
            pltpu.VMEM((K, W), jnp.float32),
            pltpu.VMEM_SHARED((ACC_ROWS, W), jnp.float32),
            pltpu.SemaphoreType.DMA,
        ],
    )
    def k(h_hbm, src_hbm, dst_hbm, astart_hbm, estart_hbm, eend_hbm,
          out_hbm, sidx, didx, av, sv, ev, rows, zbuf, acc, sem):
        cid = lax.axis_index("c")
        sid = lax.axis_index("s")
        wid = sid * NC + cid

        def zrow(i, carry):
            for j in range(W // 16):
                zbuf[i, pl.ds(j * 16, 16)] = jnp.zeros((16,), jnp.float32)
            return carry
        lax.fori_loop(0, K, zrow, 0)
        for t in range(acc_per_tile // K):
            pltpu.sync_copy(zbuf, acc.at[pl.ds(sid * acc_per_tile + t * K, K)])
        plsc.subcore_barrier()

        pltpu.sync_copy(astart_hbm, av)
        pltpu.sync_copy(estart_hbm, sv)
        pltpu.sync_copy(eend_hbm, ev)
        a0 = _extract32(av, wid)
        s0 = _extract32(sv, wid)
        e0 = _extract32(ev, wid)
        nwin = (e0 - a0 + K - 1) // K
        li = lax.broadcasted_iota(jnp.int32, (16,), 0)

        def body(j, carry):
            off = pl.multiple_of(a0 + j * K, 8)
            pltpu.sync_copy(src_hbm.at[pl.ds(off, K)], sidx)
            pltpu.sync_copy(dst_hbm.at[pl.ds(off, K)], didx)
            for sub in range(K // 16):
                gpos = li + (off + sub * 16)
                d = didx[pl.ds(sub * 16, 16)]
                keep = (gpos >= s0) & (gpos < e0)
                didx[pl.ds(sub * 16, 16)] = jnp.where(
                    keep, d, N + (wid * 7 + sub) % (ACC_ROWS - N))
            pltpu.async_copy(h_hbm.at[sidx], rows, sem).wait()
            pltpu.sync_copy(rows, acc.at[didx], add=True)
            return carry
        lax.fori_loop(0, nwin, body, 0)
        plsc.subcore_barrier()

        pltpu.sync_copy(acc.at[pl.ds(sid * acc_per_tile, acc_per_tile)],
                        out_hbm.at[cid, pl.ds(sid * acc_per_tile,
                                              acc_per_tile)])

    return k


def _pad_width(w):
    for wp in (32, 64, 80, 96, 128):
        if w <= wp:
            return wp
    raise ValueError(w)


def _aggregate(h, idxs):
    src_full, dst_full, astarts, estarts, eends = idxs
    w = h.shape[1]
    wp = _pad_width(w)
    hp = jnp.pad(h, ((0, 0), (0, wp - w))) if wp != w else h
    out = _seg_kernel(wp)(hp, src_full, dst_full, astarts, estarts, eends)
    return out[:, :N, :w]


@functools.lru_cache(maxsize=None)
def _mlp_kernel(win, oc):
    def body(agg_ref, w1_ref, b1_ref, w2_ref, b2_ref, g_ref, bt_ref,
             out_ref):
        u = agg_ref[0] + agg_ref[1]
        t = jnp.dot(u, w1_ref[...], preferred_element_type=jnp.float32)
        t = jnp.maximum(t + b1_ref[...], 0.0)
        t = jnp.dot(t, w2_ref[...], preferred_element_type=jnp.float32)
        t = t + b2_ref[...]
        m = jnp.mean(t, axis=0, keepdims=True)
        v = jnp.mean((t - m) * (t - m), axis=0, keepdims=True)
        t = (t - m) / jnp.sqrt(v + 1e-5) * g_ref[...] + bt_ref[...]
        out_ref[...] = jnp.maximum(t, 0.0)

    return pl.pallas_call(
        body, out_shape=jax.ShapeDtypeStruct((N, oc), jnp.float32))


def _mlp(agg2, p):
    w1 = p["W1"]
    return _mlp_kernel(w1.shape[0], w1.shape[1])(
        agg2, w1, p["b1"].reshape(1, -1), p["W2"], p["b2"].reshape(1, -1),
        p["gamma"].reshape(1, -1), p["beta"].reshape(1, -1))


@functools.lru_cache(maxsize=None)
def _pool_kernel(f):
    def body(h_ref, b_ref, wc_ref, bc_ref, out_ref):
        onehot = (lax.broadcasted_iota(jnp.int32, (NG, N), 0)
                  == b_ref[...]).astype(jnp.float32)
        sums = jnp.dot(onehot, h_ref[...],
                       preferred_element_type=jnp.float32,
                       precision=lax.Precision.HIGHEST)
        counts = jnp.sum(onehot, axis=1, keepdims=True)
        pooled = sums / jnp.maximum(counts, 1.0)
        out_ref[...] = jnp.dot(pooled, wc_ref[...],
                               preferred_element_type=jnp.float32) + bc_ref[...]

    return pl.pallas_call(
        body, out_shape=jax.ShapeDtypeStruct((NG, f), jnp.float32))


def kernel(x, edge_index, batch, params, Wc, bc):
    src, dst = edge_index[0], edge_index[1]
    self_ids = jnp.arange(N, dtype=jnp.int32)
    all_src = jnp.concatenate([src, self_ids])
    all_dst = jnp.concatenate([dst, self_ids])
    order = jnp.argsort(all_dst, stable=True)
    n_pad = E_ALL - E_SELF
    pad_ids = jnp.arange(n_pad, dtype=jnp.int32)
    src_full = jnp.concatenate([all_src[order], pad_ids % 16])
    dst_full = jnp.concatenate([all_dst[order],
                                N + pad_ids % (ACC_ROWS - N)])
    row_starts = jnp.arange(0, ACC_ROWS, ROWS_PER_W, dtype=jnp.int32)
    estarts = jnp.searchsorted(dst_full[:E_PAD], row_starts,
                               side="left").astype(jnp.int32)
    eends = jnp.concatenate(
        [estarts[1:], jnp.array([E_PAD], dtype=jnp.int32)])
    astarts = estarts & ~jnp.int32(7)
    idxs = (src_full, dst_full, astarts, estarts, eends)

    agg = lambda h: _aggregate(h, idxs)

    idx = 0
    h = _mlp(agg(x), params[idx]); idx += 1
    for i, nl in enumerate(_BLOCK_CONFIG):
        feat_aggs = [agg(h)]
        for j in range(nl):
            cat = jnp.concatenate(feat_aggs, axis=2)
            t = _mlp(cat, params[idx]); idx += 1
            t = _mlp(agg(t), params[idx]); idx += 1
            feat_aggs.append(agg(t))
        h = _mlp(jnp.concatenate(feat_aggs, axis=2), params[idx]); idx += 1

    return _pool_kernel(Wc.shape[1])(
        h, batch.reshape(1, N), Wc, bc.reshape(1, -1))


# speedup vs baseline: 9.6721x; 1.7022x over previous
import functools

import jax
import jax.numpy as jnp
from jax import lax
from jax.experimental import pallas as pl
from jax.experimental.pallas import tpu as pltpu
from jax.experimental.pallas import tpu_sc as plsc

N = 10000
E = 320000
NG = 64
ACC_ROWS = 10240
NC, NS = 2, 16
NW = NC * NS
K = 128
ROWS_PER_W = ACC_ROWS // NW
E_SELF = E + N
E_PAD = ((E_SELF + NW * K - 1) // (NW * K)) * (NW * K)
CHUNK = 32
E_ALL = E_PAD + CHUNK * K

_BLOCK_CONFIG = (3, 3, 3, 3)


def _extract32(vref, pos):
    li = lax.broadcasted_iota(jnp.int32, (16,), 0)
    lo = vref[pl.ds(0, 16)]
    hi = vref[pl.ds(16, 16)]
    p0 = jnp.where(li == pos, lo, 0)
    p1 = jnp.where(li + 16 == pos, hi, 0)
    return jnp.sum(p0) + jnp.sum(p1)


@functools.lru_cache(maxsize=None)
def _seg_kernel(W):
    mesh = plsc.VectorSubcoreMesh(core_axis_name="c", subcore_axis_name="s")
    acc_per_tile = ACC_ROWS // NS

    @functools.partial(
        pl.kernel,
        mesh=mesh,
        out_type=jax.ShapeDtypeStruct((NC, ACC_ROWS, W), jnp.float32),
        compiler_params=pltpu.CompilerParams(use_tc_tiling_on_sc=False,
                                             needs_layout_passes=False),
        scratch_types=[
            pltpu.VMEM((CHUNK * K,), jnp.int32),
            pltpu.VMEM((CHUNK * K,), jnp.int32),
            pltpu.VMEM((K,), jnp.int32),
            pltpu.VMEM((32,), jnp.int32),
            pltpu.VMEM((32,), jnp.int32),
            pltpu.VMEM((32,), jnp.int32),
            pltpu.VMEM((K, W), jnp.float32),
            pltpu.VMEM((K, W), jnp.float32),
            pltpu.VMEM((K, W), jnp.float32),
            pltpu.VMEM_SHARED((ACC_ROWS, W), jnp.float32),
            pltpu.SemaphoreType.DMA,
            pltpu.SemaphoreType.DMA,
        ],
    )
    def k(h_hbm, src_hbm, dst_hbm, astart_hbm, estart_hbm, eend_hbm,
          out_hbm, schunk, dchunk, didx, av, sv, ev, rows0, rows1, zbuf,
          acc, sem0, sem1):
        cid = lax.axis_index("c")
        sid = lax.axis_index("s")
        wid = sid * NC + cid
        rows = (rows0, rows1)
        sems = (sem0, sem1)

        def zrow(i, carry):
            for j in range(W // 16):
                zbuf[i, pl.ds(j * 16, 16)] = jnp.zeros((16,), jnp.float32)
            return carry
        lax.fori_loop(0, K, zrow, 0)
        for t in range(acc_per_tile // K):
            pltpu.sync_copy(zbuf, acc.at[pl.ds(sid * acc_per_tile + t * K, K)])
        plsc.subcore_barrier()

        pltpu.sync_copy(astart_hbm, av)
        pltpu.sync_copy(estart_hbm, sv)
        pltpu.sync_copy(eend_hbm, ev)
        a0 = _extract32(av, wid)
        s0 = _extract32(sv, wid)
        e0 = _extract32(ev, wid)
        nwin = (e0 - a0 + K - 1) // K
        nchunk = (nwin + CHUNK - 1) // CHUNK
        li = lax.broadcasted_iota(jnp.int32, (16,), 0)

        def chunk_body(c, carry):
            coff = pl.multiple_of(a0 + c * (CHUNK * K), 8)
            pltpu.sync_copy(src_hbm.at[pl.ds(coff, CHUNK * K)], schunk)
            pltpu.sync_copy(dst_hbm.at[pl.ds(coff, CHUNK * K)], dchunk)
            m = jnp.minimum(CHUNK, nwin - c * CHUNK)
            pltpu.async_copy(h_hbm.at[schunk.at[pl.ds(0, K)]], rows0, sem0)

            def win_pair(i, carry2):
                for b in range(2):
                    j = i * 2 + b

                    @pl.when(j < m)
                    def _():
                        @pl.when(j + 1 < m)
                        def _():
                            pltpu.async_copy(
                                h_hbm.at[schunk.at[pl.ds((j + 1) * K, K)]],
                                rows[(b + 1) % 2], sems[(b + 1) % 2])
                        for sub in range(K // 16):
                            gpos = li + (coff + j * K + sub * 16)
                            d = dchunk[pl.ds(j * K + sub * 16, 16)]
                            keep = (gpos >= s0) & (gpos < e0)
                            didx[pl.ds(sub * 16, 16)] = jnp.where(
                                keep, d, N + (wid * 7 + sub) % (ACC_ROWS - N))
                        pltpu.make_async_copy(
                            h_hbm.at[pl.ds(0, K)], rows[b], sems[b]).wait()
                        pltpu.sync_copy(rows[b], acc.at[didx], add=True)
                return carry2
            lax.fori_loop(0, (m + 1) // 2, win_pair, 0)
            return carry
        lax.fori_loop(0, nchunk, chunk_body, 0)
        plsc.subcore_barrier()

        pltpu.sync_copy(acc.at[pl.ds(sid * acc_per_tile, acc_per_tile)],
                        out_hbm.at[cid, pl.ds(sid * acc_per_tile,
                                              acc_per_tile)])

    return k


def _pad_width(w):
    for wp in (32, 64, 80, 96):
        if w <= wp:
            return wp
    raise ValueError(w)


def _aggregate(h, idxs):
    src_full, dst_full, astarts, estarts, eends = idxs
    w = h.shape[1]
    if w > 96:
        h0, h1 = h[:, :w // 2], h[:, w // 2:]
        return jnp.concatenate(
            [_aggregate(h0, idxs), _aggregate(h1, idxs)], axis=2)
    wp = _pad_width(w)
    hp = jnp.pad(h, ((0, 0), (0, wp - w))) if wp != w else h
    out = _seg_kernel(wp)(hp, src_full, dst_full, astarts, estarts, eends)
    return out[:, :N, :w]


@functools.lru_cache(maxsize=None)
def _mlp_kernel(win, oc):
    def body(agg_ref, w1_ref, b1_ref, w2_ref, b2_ref, g_ref, bt_ref,
             out_ref):
        u = agg_ref[0] + agg_ref[1]
        t = jnp.dot(u, w1_ref[...], preferred_element_type=jnp.float32)
        t = jnp.maximum(t + b1_ref[...], 0.0)
        t = jnp.dot(t, w2_ref[...], preferred_element_type=jnp.float32)
        t = t + b2_ref[...]
        m = jnp.mean(t, axis=0, keepdims=True)
        v = jnp.mean((t - m) * (t - m), axis=0, keepdims=True)
        t = (t - m) / jnp.sqrt(v + 1e-5) * g_ref[...] + bt_ref[...]
        out_ref[...] = jnp.maximum(t, 0.0)

    return pl.pallas_call(
        body, out_shape=jax.ShapeDtypeStruct((N, oc), jnp.float32))


def _mlp(agg2, p):
    w1 = p["W1"]
    return _mlp_kernel(w1.shape[0], w1.shape[1])(
        agg2, w1, p["b1"].reshape(1, -1), p["W2"], p["b2"].reshape(1, -1),
        p["gamma"].reshape(1, -1), p["beta"].reshape(1, -1))


@functools.lru_cache(maxsize=None)
def _pool_kernel(f):
    def body(h_ref, b_ref, wc_ref, bc_ref, out_ref):
        onehot = (lax.broadcasted_iota(jnp.int32, (NG, N), 0)
                  == b_ref[...]).astype(jnp.float32)
        sums = jnp.dot(onehot, h_ref[...],
                       preferred_element_type=jnp.float32,
                       precision=lax.Precision.HIGHEST)
        counts = jnp.sum(onehot, axis=1, keepdims=True)
        pooled = sums / jnp.maximum(counts, 1.0)
        out_ref[...] = jnp.dot(pooled, wc_ref[...],
                               preferred_element_type=jnp.float32) + bc_ref[...]

    return pl.pallas_call(
        body, out_shape=jax.ShapeDtypeStruct((NG, f), jnp.float32))


def kernel(x, edge_index, batch, params, Wc, bc):
    src, dst = edge_index[0], edge_index[1]
    self_ids = jnp.arange(N, dtype=jnp.int32)
    all_src = jnp.concatenate([src, self_ids])
    all_dst = jnp.concatenate([dst, self_ids])
    order = jnp.argsort(all_dst, stable=True)
    n_pad = E_ALL - E_SELF
    pad_ids = jnp.arange(n_pad, dtype=jnp.int32)
    src_full = jnp.concatenate([all_src[order], pad_ids % 16])
    dst_full = jnp.concatenate([all_dst[order],
                                N + pad_ids % (ACC_ROWS - N)])
    row_starts = jnp.arange(0, ACC_ROWS, ROWS_PER_W, dtype=jnp.int32)
    estarts = jnp.searchsorted(dst_full[:E_PAD], row_starts,
                               side="left").astype(jnp.int32)
    eends = jnp.concatenate(
        [estarts[1:], jnp.array([E_PAD], dtype=jnp.int32)])
    astarts = estarts & ~jnp.int32(7)
    idxs = (src_full, dst_full, astarts, estarts, eends)

    agg = lambda h: _aggregate(h, idxs)

    idx = 0
    h = _mlp(agg(x), params[idx]); idx += 1
    for i, nl in enumerate(_BLOCK_CONFIG):
        feat_aggs = [agg(h)]
        for j in range(nl):
            cat = jnp.concatenate(feat_aggs, axis=2)
            t = _mlp(cat, params[idx]); idx += 1
            t = _mlp(agg(t), params[idx]); idx += 1
            feat_aggs.append(agg(t))
        h = _mlp(jnp.concatenate(feat_aggs, axis=2), params[idx]); idx += 1

    return _pool_kernel(Wc.shape[1])(
        h, batch.reshape(1, N), Wc, bc.reshape(1, -1))
